# Initial kernel scaffold; baseline (speedup 1.0000x reference)
import jax
import jax.numpy as jnp
from jax.experimental import pallas as pl


def kernel(char_ids, seg_ids, char_table, seg_table):
    raise NotImplementedError("write your pallas kernel here")


# speedup vs baseline: 1.2519x; 1.2519x over previous
import functools

import jax
import jax.numpy as jnp
from jax import lax
from jax.experimental import pallas as pl
from jax.experimental.pallas import tpu as pltpu
from jax.experimental.pallas import tpu_sc as plsc

N = 4096 * 200
CHAR_DIM = 64
SEG_DIM = 16
OUT_DIM = CHAR_DIM + SEG_DIM
NC, NS = 2, 16
NW = NC * NS
PER_W = N // NW
IDX_W = 128
K = 8
T = K * IDX_W
STEPS = PER_W // T


def _sc_gather(cid2, sid2, char_table, seg_table):
    mesh = plsc.VectorSubcoreMesh(core_axis_name="c", subcore_axis_name="s")

    @functools.partial(
        pl.kernel,
        out_type=jax.ShapeDtypeStruct((N, OUT_DIM), jnp.float32),
        mesh=mesh,
        compiler_params=pltpu.CompilerParams(use_tc_tiling_on_sc=False),
        scratch_types=[
            pltpu.VMEM((K, IDX_W), jnp.int32),
            pltpu.VMEM((K, IDX_W), jnp.int32),
            pltpu.VMEM((T, CHAR_DIM), jnp.float32),
            pltpu.VMEM((T, SEG_DIM), jnp.float32),
            pltpu.SemaphoreType.DMA,
        ],
    )
    def k(cid_hbm, sid_hbm, ctab_hbm, stab_hbm, out_hbm,
          cidx, sidx, crows, srows, sem):
        wid = lax.axis_index("s") * NC + lax.axis_index("c")
        idx_row0 = wid * (PER_W // IDX_W)

        def body(i, carry):
            r = idx_row0 + i * K
            pltpu.sync_copy(cid_hbm.at[pl.ds(r, K)], cidx)
            pltpu.sync_copy(sid_hbm.at[pl.ds(r, K)], sidx)
            copies = []
            for j in range(K):
                copies.append(pltpu.async_copy(
                    ctab_hbm.at[cidx.at[j]],
                    crows.at[pl.ds(j * IDX_W, IDX_W)], sem))
                copies.append(pltpu.async_copy(
                    stab_hbm.at[sidx.at[j]],
                    srows.at[pl.ds(j * IDX_W, IDX_W)], sem))
            for c in copies:
                c.wait()
            base = wid * PER_W + i * T
            pltpu.sync_copy(crows,
                            out_hbm.at[pl.ds(base, T), pl.ds(0, CHAR_DIM)])
            pltpu.sync_copy(srows,
                            out_hbm.at[pl.ds(base, T),
                                       pl.ds(CHAR_DIM, SEG_DIM)])
            return carry

        lax.fori_loop(0, STEPS, body, 0)

    return k(cid2, sid2, char_table, seg_table)


def kernel(char_ids, seg_ids, char_table, seg_table):
    B, L = char_ids.shape
    cid2 = char_ids.reshape(N // IDX_W, IDX_W).astype(jnp.int32)
    sid2 = seg_ids.reshape(N // IDX_W, IDX_W).astype(jnp.int32)
    out = _sc_gather(cid2, sid2, char_table, seg_table)
    return out.reshape(B, L, OUT_DIM)


# speedup vs baseline: 7.2081x; 5.7579x over previous
import functools

import jax
import jax.numpy as jnp
from jax import lax
from jax.experimental import pallas as pl
from jax.experimental.pallas import tpu as pltpu
from jax.experimental.pallas import tpu_sc as plsc

VOCAB_CHAR = 1000
VOCAB_SEG = 4
CHAR_DIM = 64
SEG_DIM = 16
OUT_DIM = CHAR_DIM + SEG_DIM
FUSED_ROWS = VOCAB_CHAR * VOCAB_SEG

N = 4096 * 200
NC, NS = 2, 16
NW = NC * NS
PER_W = N // NW
IDX_W = 128
K = 4
T = K * IDX_W
STEPS = PER_W // T
NBUF = 2
PAIRS = STEPS // NBUF
ROWS_PER_TILE = FUSED_ROWS // NS
LANES = 16


def _sc_gather(cid2, sid2, char_table, seg_table):
    mesh = plsc.VectorSubcoreMesh(core_axis_name="c", subcore_axis_name="s")

    @functools.partial(
        pl.kernel,
        out_type=jax.ShapeDtypeStruct((N, OUT_DIM), jnp.float32),
        mesh=mesh,
        compiler_params=pltpu.CompilerParams(use_tc_tiling_on_sc=False),
        scratch_types=[
            pltpu.VMEM_SHARED((FUSED_ROWS, OUT_DIM), jnp.float32),
            pltpu.VMEM((NBUF * K, IDX_W), jnp.int32),
            pltpu.VMEM((NBUF * K, IDX_W), jnp.int32),
            pltpu.VMEM((NBUF * K, IDX_W), jnp.int32),
            pltpu.VMEM((NBUF * T, OUT_DIM), jnp.float32),
            pltpu.VMEM((2 * IDX_W, CHAR_DIM), jnp.float32),
            pltpu.VMEM((VOCAB_SEG, SEG_DIM), jnp.float32),
            pltpu.SemaphoreType.DMA,
            pltpu.SemaphoreType.DMA,
            pltpu.SemaphoreType.DMA,
            pltpu.SemaphoreType.DMA,
        ],
    )
    def k(cid_hbm, sid_hbm, ctab_hbm, stab_hbm, out_hbm,
          ftab, cidx, sidx, fidx, frows, ctmp, stab_v,
          gsem0, gsem1, wsem0, wsem1):
        sid = lax.axis_index("s")
        wid = sid * NC + lax.axis_index("c")
        iot = lax.iota(jnp.int32, LANES)

        r0 = sid * ROWS_PER_TILE
        for j in range(2):
            for l in range(IDX_W // LANES):
                rvec = r0 + (j * IDX_W + l * LANES) + iot
                cvec = jnp.minimum(rvec >> 2, VOCAB_CHAR - 1)
                fidx[j, pl.ds(l * LANES, LANES)] = cvec
        pltpu.sync_copy(stab_hbm, stab_v)
        g0 = pltpu.async_copy(ctab_hbm.at[fidx.at[0]],
                              ctmp.at[pl.ds(0, IDX_W)], gsem0)
        g1 = pltpu.async_copy(ctab_hbm.at[fidx.at[1]],
                              ctmp.at[pl.ds(IDX_W, IDX_W)], gsem0)
        g0.wait()
        g1.wait()

        def asm_body(i, carry):
            for c4 in range(CHAR_DIM // LANES):
                frows[i, pl.ds(c4 * LANES, LANES)] = (
                    ctmp[i, pl.ds(c4 * LANES, LANES)])
            s = (r0 + i) & (VOCAB_SEG - 1)
            frows[i, pl.ds(CHAR_DIM, SEG_DIM)] = stab_v[s, pl.ds(0, SEG_DIM)]
            return carry

        lax.fori_loop(0, ROWS_PER_TILE, asm_body, 0)
        pltpu.sync_copy(frows.at[pl.ds(0, ROWS_PER_TILE)],
                        ftab.at[pl.ds(r0, ROWS_PER_TILE)])
        plsc.subcore_barrier()

        gsems = (gsem0, gsem1)
        wsems = (wsem0, wsem1)
        idx_row0 = wid * (PER_W // IDX_W)

        def pair_body(p, carry):
            gathers = []
            for buf in range(NBUF):
                i = p * NBUF + buf

                @pl.when(p > 0)
                def _drain():
                    prev_base = wid * PER_W + (i - NBUF) * T
                    pltpu.make_async_copy(
                        frows.at[pl.ds(buf * T, T)],
                        out_hbm.at[pl.ds(prev_base, T)],
                        wsems[buf]).wait()

                r = idx_row0 + i * K
                pltpu.sync_copy(cid_hbm.at[pl.ds(r, K)],
                                cidx.at[pl.ds(buf * K, K)])
                pltpu.sync_copy(sid_hbm.at[pl.ds(r, K)],
                                sidx.at[pl.ds(buf * K, K)])
                for j in range(K):
                    for l in range(IDX_W // LANES):
                        sl = pl.ds(l * LANES, LANES)
                        cv = cidx[buf * K + j, sl]
                        sv = sidx[buf * K + j, sl]
                        fidx[buf * K + j, sl] = (cv << 2) + sv
                bg = []
                for j in range(K):
                    bg.append(pltpu.async_copy(
                        ftab.at[fidx.at[buf * K + j]],
                        frows.at[pl.ds(buf * T + j * IDX_W, IDX_W)],
                        gsems[buf]))
                gathers.append(bg)
            for buf in range(NBUF):
                i = p * NBUF + buf
                for g in gathers[buf]:
                    g.wait()
                base = wid * PER_W + i * T
                pltpu.async_copy(frows.at[pl.ds(buf * T, T)],
                                 out_hbm.at[pl.ds(base, T)], wsems[buf])
            return carry

        lax.fori_loop(0, PAIRS, pair_body, 0)
        for buf in range(NBUF):
            base = wid * PER_W + (STEPS - NBUF + buf) * T
            pltpu.make_async_copy(frows.at[pl.ds(buf * T, T)],
                                  out_hbm.at[pl.ds(base, T)],
                                  wsems[buf]).wait()

    return k(cid2, sid2, char_table, seg_table)


def kernel(char_ids, seg_ids, char_table, seg_table):
    B, L = char_ids.shape
    cid2 = char_ids.reshape(N // IDX_W, IDX_W).astype(jnp.int32)
    sid2 = seg_ids.reshape(N // IDX_W, IDX_W).astype(jnp.int32)
    out = _sc_gather(cid2, sid2, char_table, seg_table)
    return out.reshape(B, L, OUT_DIM)
